# baseline (speedup 1.0000x reference)
import functools

import jax
import jax.numpy as jnp
from jax import lax
from jax.experimental import pallas as pl
from jax.experimental.pallas import tpu as pltpu
from jax.experimental.pallas import tpu_sc as plsc

_VOCAB = 100000
_DIM = 128
_B, _S = 4, 4096
_NUM_LAYERS = 12

_NTOT = _B * _S
_NC, _NS = 2, 16
_NW = _NC * _NS
_PER_W = _NTOT // _NW
_CHUNK = 128
_ROWS_W = _PER_W // _CHUNK
_NBUF = 4
_NGIF = 2

_BLK = 1024


def _gather_one(idx2d, table):
    mesh = plsc.VectorSubcoreMesh(core_axis_name="c", subcore_axis_name="s")

    @functools.partial(
        pl.kernel,
        mesh=mesh,
        out_type=jax.ShapeDtypeStruct((_NTOT, _DIM), jnp.float32),
        scratch_types=[
            pltpu.VMEM((_ROWS_W, _CHUNK), jnp.int32),
            pltpu.VMEM((_NBUF, _CHUNK, _DIM), jnp.float32),
            pltpu.SemaphoreType.DMA((_NBUF,)),
            pltpu.SemaphoreType.DMA((_NBUF,)),
        ],
    )
    def body(idx_hbm, tab, out, idx_v, bufs, gsem, wsem):
        wid = lax.axis_index("s") * _NC + lax.axis_index("c")
        pltpu.sync_copy(idx_hbm.at[pl.ds(wid * _ROWS_W, _ROWS_W)], idx_v)

        n = _ROWS_W

        def issue_gather(j):
            return pltpu.async_copy(
                tab.at[idx_v.at[j]], bufs.at[j % _NBUF], gsem.at[j % _NBUF])

        gh = [None] * n
        wh = [None] * n
        for j in range(_NGIF):
            gh[j] = issue_gather(j)
        for j in range(n):
            gh[j].wait()
            row0 = wid * _PER_W + j * _CHUNK
            wh[j] = pltpu.async_copy(
                bufs.at[j % _NBUF], out.at[pl.ds(row0, _CHUNK)],
                wsem.at[j % _NBUF])
            nxt = j + _NGIF
            if nxt < n:
                if nxt >= _NBUF:
                    wh[nxt - _NBUF].wait()
                gh[nxt] = issue_gather(nxt)
        for j in range(max(0, n - _NBUF), n):
            wh[j].wait()

    return body(idx2d, table)


def _dup3(src):
    def body(x_ref, a_ref, b_ref, c_ref):
        v = x_ref[...]
        a_ref[...] = v
        b_ref[...] = v
        c_ref[...] = v

    spec = pl.BlockSpec((_BLK, _DIM), lambda i: (i, 0))
    return pl.pallas_call(
        body,
        grid=(_NTOT // _BLK,),
        in_specs=[spec],
        out_specs=[spec] * 3,
        out_shape=[jax.ShapeDtypeStruct((_NTOT, _DIM), jnp.float32)] * 3,
    )(src)


def kernel(input_seq, W0, W1, W2):
    idx2d = input_seq.reshape(_NTOT // _CHUNK, _CHUNK)
    uniq = [_gather_one(idx2d, w) for w in (W0, W1, W2)]
    dups = [_dup3(o) for o in uniq]
    outs = list(uniq)
    for k in range(3):
        for t in range(3):
            outs.append(dups[t][k])
    return tuple(o.reshape(_B, _S, _DIM) for o in outs)
